# baseline (speedup 1.0000x reference)
import jax
import jax.numpy as jnp
from jax.experimental import pallas as pl


def _fiota(shape, dim):
    return jax.lax.broadcasted_iota(jnp.int32, shape, dim).astype(jnp.float32)


def _dot3(a, b):
    ah = a.astype(jnp.bfloat16)
    al = (a - ah.astype(jnp.float32)).astype(jnp.bfloat16)
    bh = b.astype(jnp.bfloat16)
    bl = (b - bh.astype(jnp.float32)).astype(jnp.bfloat16)
    d = lambda x, y: jnp.dot(x, y, preferred_element_type=jnp.float32)
    return d(ah, bh) + d(al, bh) + d(ah, bl)


def _slice_body(gridt_ref, guide_ref, out_ref, *, C, D, GH, GW, H, W, HB, ZMIN, DZ):
    hb = pl.program_id(1)

    h_idx = hb * HB + _fiota((GW, HB), 1)
    hg = h_idx / (H - 1) * 2.0 - 1.0
    xc = (hg + 1.0) * 0.5 * (GW - 1)
    xrow = _fiota((GW, HB), 0)
    At = jnp.maximum(0.0, 1.0 - jnp.abs(xc - xrow))

    G1 = _dot3(gridt_ref[0], At)
    G1 = G1.reshape(C * DZ, GH, HB)
    G1 = jnp.swapaxes(G1, 1, 2).reshape(C * DZ * HB, GH)

    w_idx = _fiota((GH, W), 1)
    wg = w_idx / (W - 1) * 2.0 - 1.0
    yc = (wg + 1.0) * 0.5 * (GH - 1)
    yrow = _fiota((GH, W), 0)
    Bt = jnp.maximum(0.0, 1.0 - jnp.abs(yc - yrow))

    P = _dot3(G1, Bt)
    P = P.reshape(C, DZ, HB, W)

    g = guide_ref[0, 0]
    zc = (g + 1.0) * 0.5 * (D - 1)
    acc = jnp.zeros((C, HB, W), dtype=jnp.float32)
    for z in range(DZ):
        m = jnp.maximum(0.0, 1.0 - jnp.abs(zc - float(ZMIN + z)))
        acc = acc + P[:, z] * m[None]
    out_ref[0] = acc


def _sc_slice(gridz, gm3, *, C, D, ZMIN, DZ, GH, GW):
    import functools
    from jax import lax
    from jax.experimental.pallas import tpu as pltpu
    from jax.experimental.pallas import tpu_sc as plsc

    N, H, W = gm3.shape
    info = plsc.get_sparse_core_info()
    NC, NS = info.num_cores, info.num_subcores
    NW = NC * NS
    hs_w = H // NW
    BR = 4
    bpn = hs_w // BR
    NV = W // 16
    mesh = plsc.VectorSubcoreMesh(core_axis_name="c", subcore_axis_name="s")
    CZ = C * DZ
    zhi = float(D - 1)

    @functools.partial(
        pl.kernel, mesh=mesh,
        out_type=jax.ShapeDtypeStruct((N, C, H, W), jnp.float32),
        compiler_params=pltpu.CompilerParams(needs_layout_passes=False),
        scratch_types=[
            pltpu.VMEM((CZ * GH * GW,), jnp.float32),
            pltpu.VMEM((CZ * GH,), jnp.float32),
            pltpu.VMEM((hs_w, W), jnp.float32),
            pltpu.VMEM((2, C, BR, W), jnp.float32),
            pltpu.VMEM((W,), jnp.int32),
            pltpu.VMEM((W,), jnp.int32),
            pltpu.VMEM((W,), jnp.float32),
            pltpu.VMEM((W,), jnp.float32),
            pltpu.SemaphoreType.DMA((2,)),
        ],
    )
    def sck(gridz_hbm, gm3_hbm, out_hbm, gbuf, rbuf, gdbuf, obuf,
            ybuf, dybuf, wy0buf, wy1buf, sems):
        wid = lax.axis_index("s") * NC + lax.axis_index("c")
        iota16 = lax.broadcasted_iota(jnp.int32, (16,), 0)
        iota16x = iota16 * GW

        def build_tables(j, _):
            wv = (j * 16 + iota16).astype(jnp.float32)
            wgn = wv / (W - 1) * 2.0 - 1.0
            yf = (wgn + 1.0) * 0.5 * (GH - 1)
            y0i = yf.astype(jnp.int32)
            wy1 = yf - y0i.astype(jnp.float32)
            dy = jnp.minimum(y0i + 1, GH - 1) - y0i
            sl = pl.ds(j * 16, 16)
            ybuf[sl] = y0i
            dybuf[sl] = dy
            wy0buf[sl] = 1.0 - wy1
            wy1buf[sl] = wy1
            return 0
        lax.fori_loop(0, NV, build_tables, 0)

        h_w0 = wid * hs_w

        def do_block(n, blk, gblk):
            h0 = h_w0 + blk * BR
            p = lax.rem(gblk, 2)
            @pl.when(gblk >= 2)
            def _():
                for c in range(C):
                    pltpu.make_async_copy(
                        obuf.at[p, c],
                        out_hbm.at[n, c, pl.ds(h0, BR), :],
                        sems.at[p]).wait()

            def do_row(r4, _):
                h = h0 + r4
                hf = (jnp.zeros((16,), jnp.int32) + h).astype(jnp.float32)
                hgn = hf / (H - 1) * 2.0 - 1.0
                xf = (hgn + 1.0) * 0.5 * (GW - 1)
                x0 = xf.astype(jnp.int32)
                wx1 = xf - x0.astype(jnp.float32)
                wx0 = 1.0 - wx1
                dx = jnp.minimum(x0 + 1, GW - 1) - x0

                @plsc.parallel_loop(0, CZ, unroll=4)
                def fold_x(k):
                    idx = k * (GH * GW) + iota16x + x0
                    g0 = plsc.load_gather(gbuf, [idx])
                    g1 = plsc.load_gather(gbuf, [idx + dx])
                    rbuf[pl.ds(k * 16, 16)] = g0 * wx0 + g1 * wx1

                lrow = blk * BR + r4

                @plsc.parallel_loop(0, NV, unroll=6)
                def do_vec(j):
                    sl = pl.ds(j * 16, 16)
                    g = gdbuf[lrow, sl]
                    zc = (g + 1.0) * 0.5 * (D - 1)
                    y0v = ybuf[sl]
                    dyv = dybuf[sl]
                    wy0v = wy0buf[sl]
                    wy1v = wy1buf[sl]
                    y1v = y0v + dyv
                    ms = [jnp.maximum(0.0, 1.0 - jnp.abs(zc - float(ZMIN + zr)))
                          for zr in range(DZ)]
                    for c in range(C):
                        acc = None
                        for zr in range(DZ):
                            rv = rbuf[pl.ds((c * DZ + zr) * GH, 16)]
                            t = (rv.at[y0v].get(mode="promise_in_bounds") * wy0v
                                 + rv.at[y1v].get(mode="promise_in_bounds") * wy1v)
                            acc = t * ms[zr] if acc is None else acc + t * ms[zr]
                        obuf[p, c, r4, sl] = acc
                return 0
            lax.fori_loop(0, BR, do_row, 0)

            for c in range(C):
                pltpu.async_copy(
                    obuf.at[p, c],
                    out_hbm.at[n, c, pl.ds(h0, BR), :],
                    sems.at[p])

        def do_n(n, _):
            pltpu.sync_copy(gridz_hbm.at[n], gbuf)
            pltpu.sync_copy(gm3_hbm.at[n, pl.ds(h_w0, hs_w), :], gdbuf)
            def blk_body(blk, _):
                do_block(n, blk, n * bpn + blk)
                return 0
            lax.fori_loop(0, bpn, blk_body, 0)
            return 0
        lax.fori_loop(0, N, do_n, 0)

        for p in range(2):
            for c in range(C):
                pltpu.make_async_copy(
                    obuf.at[p, c],
                    out_hbm.at[N - 1, c, pl.ds(0, BR), :],
                    sems.at[p]).wait()

    return sck(gridz, gm3)


def kernel(bilateral_grid, guidemap):
    N, C, D, GH, GW = bilateral_grid.shape
    _, _, H, W = guidemap.shape
    ZMIN = (D - 1) // 2
    DZ = D - ZMIN
    gridz = bilateral_grid[:, :, ZMIN:].reshape(N, C * DZ * GH * GW)
    gm3 = guidemap.reshape(N, H, W)
    return _sc_slice(gridz, gm3, C=C, D=D, ZMIN=ZMIN, DZ=DZ, GH=GH, GW=GW)


def _kernel_tc(bilateral_grid, guidemap):
    N, C, D, GH, GW = bilateral_grid.shape
    _, _, H, W = guidemap.shape
    HB = 64
    ZMIN = (D - 1) // 2
    DZ = D - ZMIN
    gridt = bilateral_grid[:, :, ZMIN:].reshape(N, C * DZ * GH, GW)

    import functools
    body = functools.partial(_slice_body, C=C, D=D, GH=GH, GW=GW, H=H, W=W,
                             HB=HB, ZMIN=ZMIN, DZ=DZ)
    from jax.experimental.pallas import tpu as pltpu
    return pl.pallas_call(
        body,
        grid=(N, H // HB),
        compiler_params=pltpu.CompilerParams(
            dimension_semantics=("parallel", "parallel")),
        in_specs=[
            pl.BlockSpec((1, C * DZ * GH, GW), lambda n, j: (n, 0, 0)),
            pl.BlockSpec((1, 1, HB, W), lambda n, j: (n, 0, j, 0)),
        ],
        out_specs=pl.BlockSpec((1, C, HB, W), lambda n, j: (n, 0, j, 0)),
        out_shape=jax.ShapeDtypeStruct((N, C, H, W), jnp.float32),
    )(gridt, guidemap)


# speedup vs baseline: 1.1902x; 1.1902x over previous
import jax
import jax.numpy as jnp
from jax.experimental import pallas as pl


def _fiota(shape, dim):
    return jax.lax.broadcasted_iota(jnp.int32, shape, dim).astype(jnp.float32)


def _dot3(a, b):
    ah = a.astype(jnp.bfloat16)
    al = (a - ah.astype(jnp.float32)).astype(jnp.bfloat16)
    bh = b.astype(jnp.bfloat16)
    bl = (b - bh.astype(jnp.float32)).astype(jnp.bfloat16)
    d = lambda x, y: jnp.dot(x, y, preferred_element_type=jnp.float32)
    return d(ah, bh) + d(al, bh) + d(ah, bl)


def _slice_body(gridt_ref, guide_ref, out_ref, *, C, D, GH, GW, H, W, HB, ZMIN, DZ):
    hb = pl.program_id(1)

    h_idx = hb * HB + _fiota((GW, HB), 1)
    hg = h_idx / (H - 1) * 2.0 - 1.0
    xc = (hg + 1.0) * 0.5 * (GW - 1)
    xrow = _fiota((GW, HB), 0)
    At = jnp.maximum(0.0, 1.0 - jnp.abs(xc - xrow))

    G1 = _dot3(gridt_ref[0], At)
    G1 = G1.reshape(C * DZ, GH, HB)
    G1 = jnp.swapaxes(G1, 1, 2).reshape(C * DZ * HB, GH)

    w_idx = _fiota((GH, W), 1)
    wg = w_idx / (W - 1) * 2.0 - 1.0
    yc = (wg + 1.0) * 0.5 * (GH - 1)
    yrow = _fiota((GH, W), 0)
    Bt = jnp.maximum(0.0, 1.0 - jnp.abs(yc - yrow))

    P = _dot3(G1, Bt)
    P = P.reshape(C, DZ, HB, W)

    g = guide_ref[0, 0]
    zc = (g + 1.0) * 0.5 * (D - 1)
    acc = jnp.zeros((C, HB, W), dtype=jnp.float32)
    for z in range(DZ):
        m = jnp.maximum(0.0, 1.0 - jnp.abs(zc - float(ZMIN + z)))
        acc = acc + P[:, z] * m[None]
    out_ref[0] = acc


def _sc_slice(gridz, gm3, *, C, D, ZMIN, DZ, GH, GW):
    import functools
    from jax import lax
    from jax.experimental.pallas import tpu as pltpu
    from jax.experimental.pallas import tpu_sc as plsc

    N, H, W = gm3.shape
    info = plsc.get_sparse_core_info()
    NC, NS = info.num_cores, info.num_subcores
    NW = NC * NS
    hs_w = H // NW
    BR = 4
    bpn = hs_w // BR
    NV = W // 16
    mesh = plsc.VectorSubcoreMesh(core_axis_name="c", subcore_axis_name="s")
    CZ = C * DZ
    zhi = float(D - 1)

    @functools.partial(
        pl.kernel, mesh=mesh,
        out_type=jax.ShapeDtypeStruct((N, C, H, W), jnp.float32),
        compiler_params=pltpu.CompilerParams(needs_layout_passes=False),
        scratch_types=[
            pltpu.VMEM((CZ * GH * GW,), jnp.float32),
            pltpu.VMEM((CZ * GH,), jnp.float32),
            pltpu.VMEM((hs_w, W), jnp.float32),
            pltpu.VMEM((2, C, BR, W), jnp.float32),
            pltpu.VMEM((W,), jnp.int32),
            pltpu.VMEM((W,), jnp.int32),
            pltpu.VMEM((W,), jnp.float32),
            pltpu.VMEM((W,), jnp.float32),
            pltpu.SemaphoreType.DMA((2,)),
        ],
    )
    def sck(gridz_hbm, gm3_hbm, out_hbm, gbuf, rbuf, gdbuf, obuf,
            ybuf, dybuf, wy0buf, wy1buf, sems):
        wid = lax.axis_index("s") * NC + lax.axis_index("c")
        iota16 = lax.broadcasted_iota(jnp.int32, (16,), 0)
        iota16x = iota16 * GW

        def build_tables(j, _):
            wv = (j * 16 + iota16).astype(jnp.float32)
            wgn = wv / (W - 1) * 2.0 - 1.0
            yf = (wgn + 1.0) * 0.5 * (GH - 1)
            y0i = yf.astype(jnp.int32)
            wy1 = yf - y0i.astype(jnp.float32)
            dy = jnp.minimum(y0i + 1, GH - 1) - y0i
            sl = pl.ds(j * 16, 16)
            ybuf[sl] = y0i
            dybuf[sl] = dy
            wy0buf[sl] = 1.0 - wy1
            wy1buf[sl] = wy1
            return 0
        lax.fori_loop(0, NV, build_tables, 0)

        h_w0 = wid * hs_w

        def do_block(n, blk, gblk):
            h0 = h_w0 + blk * BR
            p = lax.rem(gblk, 2)
            @pl.when(gblk >= 2)
            def _():
                for c in range(C):
                    pltpu.make_async_copy(
                        obuf.at[p, c],
                        out_hbm.at[n, c, pl.ds(h0, BR), :],
                        sems.at[p]).wait()

            def do_row(r4, _):
                h = h0 + r4
                hf = (jnp.zeros((16,), jnp.int32) + h).astype(jnp.float32)
                hgn = hf / (H - 1) * 2.0 - 1.0
                xf = (hgn + 1.0) * 0.5 * (GW - 1)
                x0 = xf.astype(jnp.int32)
                wx1 = xf - x0.astype(jnp.float32)
                wx0 = 1.0 - wx1
                dx = jnp.minimum(x0 + 1, GW - 1) - x0
                x0s = x0[0]
                dxs = dx[0]

                @plsc.parallel_loop(0, CZ, unroll=4)
                def fold_x(k):
                    base = (k * GW + x0s) * GH
                    g0 = gbuf[pl.ds(base, 16)]
                    g1 = gbuf[pl.ds(base + dxs * GH, 16)]
                    rbuf[pl.ds(k * 16, 16)] = g0 * wx0 + g1 * wx1

                lrow = blk * BR + r4

                @plsc.parallel_loop(0, NV, unroll=4)
                def do_vec(j):
                    sl = pl.ds(j * 16, 16)
                    g = gdbuf[lrow, sl]
                    zc = (g + 1.0) * 0.5 * (D - 1)
                    y0v = ybuf[sl]
                    dyv = dybuf[sl]
                    wy0v = wy0buf[sl]
                    wy1v = wy1buf[sl]
                    y1v = y0v + dyv
                    ms = [jnp.maximum(0.0, 1.0 - jnp.abs(zc - float(ZMIN + zr)))
                          for zr in range(DZ)]
                    for c in range(C):
                        acc = None
                        for zr in range(DZ):
                            rv = rbuf[pl.ds((c * DZ + zr) * GH, 16)]
                            t = (rv.at[y0v].get(mode="promise_in_bounds") * wy0v
                                 + rv.at[y1v].get(mode="promise_in_bounds") * wy1v)
                            acc = t * ms[zr] if acc is None else acc + t * ms[zr]
                        obuf[p, c, r4, sl] = acc
                return 0
            lax.fori_loop(0, BR, do_row, 0)

            for c in range(C):
                pltpu.async_copy(
                    obuf.at[p, c],
                    out_hbm.at[n, c, pl.ds(h0, BR), :],
                    sems.at[p])

        def do_n(n, _):
            pltpu.sync_copy(gridz_hbm.at[n], gbuf)
            pltpu.sync_copy(gm3_hbm.at[n, pl.ds(h_w0, hs_w), :], gdbuf)
            def blk_body(blk, _):
                do_block(n, blk, n * bpn + blk)
                return 0
            lax.fori_loop(0, bpn, blk_body, 0)
            return 0
        lax.fori_loop(0, N, do_n, 0)

        for p in range(2):
            for c in range(C):
                pltpu.make_async_copy(
                    obuf.at[p, c],
                    out_hbm.at[N - 1, c, pl.ds(0, BR), :],
                    sems.at[p]).wait()

    return sck(gridz, gm3)


def kernel(bilateral_grid, guidemap):
    N, C, D, GH, GW = bilateral_grid.shape
    _, _, H, W = guidemap.shape
    ZMIN = (D - 1) // 2
    DZ = D - ZMIN
    gridz = bilateral_grid[:, :, ZMIN:].transpose(0, 1, 2, 4, 3).reshape(N, C * DZ * GH * GW)
    gm3 = guidemap.reshape(N, H, W)
    return _sc_slice(gridz, gm3, C=C, D=D, ZMIN=ZMIN, DZ=DZ, GH=GH, GW=GW)


def _kernel_tc(bilateral_grid, guidemap):
    N, C, D, GH, GW = bilateral_grid.shape
    _, _, H, W = guidemap.shape
    HB = 64
    ZMIN = (D - 1) // 2
    DZ = D - ZMIN
    gridt = bilateral_grid[:, :, ZMIN:].reshape(N, C * DZ * GH, GW)

    import functools
    body = functools.partial(_slice_body, C=C, D=D, GH=GH, GW=GW, H=H, W=W,
                             HB=HB, ZMIN=ZMIN, DZ=DZ)
    from jax.experimental.pallas import tpu as pltpu
    return pl.pallas_call(
        body,
        grid=(N, H // HB),
        compiler_params=pltpu.CompilerParams(
            dimension_semantics=("parallel", "parallel")),
        in_specs=[
            pl.BlockSpec((1, C * DZ * GH, GW), lambda n, j: (n, 0, 0)),
            pl.BlockSpec((1, 1, HB, W), lambda n, j: (n, 0, j, 0)),
        ],
        out_specs=pl.BlockSpec((1, C, HB, W), lambda n, j: (n, 0, j, 0)),
        out_shape=jax.ShapeDtypeStruct((N, C, H, W), jnp.float32),
    )(gridt, guidemap)


# speedup vs baseline: 1.3506x; 1.1348x over previous
import jax
import jax.numpy as jnp
from jax.experimental import pallas as pl


def _fiota(shape, dim):
    return jax.lax.broadcasted_iota(jnp.int32, shape, dim).astype(jnp.float32)


def _dot3(a, b):
    ah = a.astype(jnp.bfloat16)
    al = (a - ah.astype(jnp.float32)).astype(jnp.bfloat16)
    bh = b.astype(jnp.bfloat16)
    bl = (b - bh.astype(jnp.float32)).astype(jnp.bfloat16)
    d = lambda x, y: jnp.dot(x, y, preferred_element_type=jnp.float32)
    return d(ah, bh) + d(al, bh) + d(ah, bl)


def _slice_body(gridt_ref, guide_ref, out_ref, *, C, D, GH, GW, H, W, HB, ZMIN, DZ):
    hb = pl.program_id(1)

    h_idx = hb * HB + _fiota((GW, HB), 1)
    hg = h_idx / (H - 1) * 2.0 - 1.0
    xc = (hg + 1.0) * 0.5 * (GW - 1)
    xrow = _fiota((GW, HB), 0)
    At = jnp.maximum(0.0, 1.0 - jnp.abs(xc - xrow))

    G1 = _dot3(gridt_ref[0], At)
    G1 = G1.reshape(C * DZ, GH, HB)
    G1 = jnp.swapaxes(G1, 1, 2).reshape(C * DZ * HB, GH)

    w_idx = _fiota((GH, W), 1)
    wg = w_idx / (W - 1) * 2.0 - 1.0
    yc = (wg + 1.0) * 0.5 * (GH - 1)
    yrow = _fiota((GH, W), 0)
    Bt = jnp.maximum(0.0, 1.0 - jnp.abs(yc - yrow))

    P = _dot3(G1, Bt)
    P = P.reshape(C, DZ, HB, W)

    g = guide_ref[0, 0]
    zc = (g + 1.0) * 0.5 * (D - 1)
    acc = jnp.zeros((C, HB, W), dtype=jnp.float32)
    for z in range(DZ):
        m = jnp.maximum(0.0, 1.0 - jnp.abs(zc - float(ZMIN + z)))
        acc = acc + P[:, z] * m[None]
    out_ref[0] = acc


def _sc_slice(gridz, gm3, *, C, D, ZMIN, DZ, GH, GW):
    import functools
    from jax import lax
    from jax.experimental.pallas import tpu as pltpu
    from jax.experimental.pallas import tpu_sc as plsc

    N, H, W = gm3.shape
    info = plsc.get_sparse_core_info()
    NC, NS = info.num_cores, info.num_subcores
    NW = NC * NS
    hs_w = H // NW
    BR = 4
    bpn = hs_w // BR
    NV = W // 16
    mesh = plsc.VectorSubcoreMesh(core_axis_name="c", subcore_axis_name="s")
    CZ = C * DZ
    zhi = float(D - 1)

    @functools.partial(
        pl.kernel, mesh=mesh,
        out_type=jax.ShapeDtypeStruct((N, C, H, W), jnp.float32),
        compiler_params=pltpu.CompilerParams(needs_layout_passes=False),
        scratch_types=[
            pltpu.VMEM((CZ * GH * GW,), jnp.float32),
            pltpu.VMEM((CZ * GH,), jnp.float32),
            pltpu.VMEM((hs_w, W), jnp.float32),
            pltpu.VMEM((2, C, BR, W), jnp.float32),
            pltpu.VMEM((W,), jnp.int32),
            pltpu.VMEM((W,), jnp.int32),
            pltpu.VMEM((W,), jnp.float32),
            pltpu.VMEM((W,), jnp.float32),
            pltpu.SemaphoreType.DMA((2,)),
        ],
    )
    def sck(gridz_hbm, gm3_hbm, out_hbm, gbuf, rbuf, gdbuf, obuf,
            ybuf, dybuf, wy0buf, wy1buf, sems):
        wid = lax.axis_index("s") * NC + lax.axis_index("c")
        iota16 = lax.broadcasted_iota(jnp.int32, (16,), 0)
        iota16x = iota16 * GW

        def build_tables(j, _):
            wv = (j * 16 + iota16).astype(jnp.float32)
            wgn = wv / (W - 1) * 2.0 - 1.0
            yf = (wgn + 1.0) * 0.5 * (GH - 1)
            y0i = yf.astype(jnp.int32)
            wy1 = yf - y0i.astype(jnp.float32)
            dy = jnp.minimum(y0i + 1, GH - 1) - y0i
            sl = pl.ds(j * 16, 16)
            ybuf[sl] = y0i
            dybuf[sl] = dy
            wy0buf[sl] = 1.0 - wy1
            wy1buf[sl] = wy1
            return 0
        lax.fori_loop(0, NV, build_tables, 0)

        h_w0 = wid * hs_w

        def do_block(n, blk, gblk):
            h0 = h_w0 + blk * BR
            p = lax.rem(gblk, 2)
            @pl.when(gblk >= 2)
            def _():
                for c in range(C):
                    pltpu.make_async_copy(
                        obuf.at[p, c],
                        out_hbm.at[n, c, pl.ds(h0, BR), :],
                        sems.at[p]).wait()

            def do_row(r4, _):
                h = h0 + r4
                hf = (jnp.zeros((16,), jnp.int32) + h).astype(jnp.float32)
                hgn = hf / (H - 1) * 2.0 - 1.0
                xf = (hgn + 1.0) * 0.5 * (GW - 1)
                x0 = xf.astype(jnp.int32)
                wx1 = xf - x0.astype(jnp.float32)
                wx0 = 1.0 - wx1
                dx = jnp.minimum(x0 + 1, GW - 1) - x0
                x0s = x0[0]
                dxs = dx[0]

                @plsc.parallel_loop(0, CZ, unroll=4)
                def fold_x(k):
                    base = (k * GW + x0s) * GH
                    g0 = gbuf[pl.ds(base, 16)]
                    g1 = gbuf[pl.ds(base + dxs * GH, 16)]
                    rbuf[pl.ds(k * 16, 16)] = g0 * wx0 + g1 * wx1

                lrow = blk * BR + r4

                @plsc.parallel_loop(0, NV, unroll=4)
                def do_vec(j):
                    sl = pl.ds(j * 16, 16)
                    g = gdbuf[lrow, sl]
                    zc = (g + 1.0) * 0.5 * (D - 1)
                    y0v = ybuf[sl]
                    dyv = dybuf[sl]
                    wy0v = wy0buf[sl]
                    wy1v = wy1buf[sl]
                    y1v = y0v + dyv
                    w0s, w1s = [], []
                    for zr in range(DZ):
                        m = jnp.maximum(0.0, 1.0 - jnp.abs(zc - float(ZMIN + zr)))
                        w0s.append(wy0v * m)
                        w1s.append(wy1v * m)
                    for c in range(C):
                        acc = None
                        for zr in range(DZ):
                            rv = rbuf[pl.ds((c * DZ + zr) * GH, 16)]
                            t = rv.at[y0v].get(mode="promise_in_bounds") * w0s[zr]
                            t = t + rv.at[y1v].get(mode="promise_in_bounds") * w1s[zr]
                            acc = t if acc is None else acc + t
                        obuf[p, c, r4, sl] = acc
                return 0
            lax.fori_loop(0, BR, do_row, 0)

            for c in range(C):
                pltpu.async_copy(
                    obuf.at[p, c],
                    out_hbm.at[n, c, pl.ds(h0, BR), :],
                    sems.at[p])

        def do_n(n, _):
            pltpu.sync_copy(gridz_hbm.at[n], gbuf)
            pltpu.sync_copy(gm3_hbm.at[n, pl.ds(h_w0, hs_w), :], gdbuf)
            def blk_body(blk, _):
                do_block(n, blk, n * bpn + blk)
                return 0
            lax.fori_loop(0, bpn, blk_body, 0)
            return 0
        lax.fori_loop(0, N, do_n, 0)

        for p in range(2):
            for c in range(C):
                pltpu.make_async_copy(
                    obuf.at[p, c],
                    out_hbm.at[N - 1, c, pl.ds(0, BR), :],
                    sems.at[p]).wait()

    return sck(gridz, gm3)


def kernel(bilateral_grid, guidemap):
    N, C, D, GH, GW = bilateral_grid.shape
    _, _, H, W = guidemap.shape
    ZMIN = (D - 1) // 2
    DZ = D - ZMIN
    gridz = bilateral_grid[:, :, ZMIN:].transpose(0, 1, 2, 4, 3).reshape(N, C * DZ * GH * GW)
    gm3 = guidemap.reshape(N, H, W)
    return _sc_slice(gridz, gm3, C=C, D=D, ZMIN=ZMIN, DZ=DZ, GH=GH, GW=GW)


def _kernel_tc(bilateral_grid, guidemap):
    N, C, D, GH, GW = bilateral_grid.shape
    _, _, H, W = guidemap.shape
    HB = 64
    ZMIN = (D - 1) // 2
    DZ = D - ZMIN
    gridt = bilateral_grid[:, :, ZMIN:].reshape(N, C * DZ * GH, GW)

    import functools
    body = functools.partial(_slice_body, C=C, D=D, GH=GH, GW=GW, H=H, W=W,
                             HB=HB, ZMIN=ZMIN, DZ=DZ)
    from jax.experimental.pallas import tpu as pltpu
    return pl.pallas_call(
        body,
        grid=(N, H // HB),
        compiler_params=pltpu.CompilerParams(
            dimension_semantics=("parallel", "parallel")),
        in_specs=[
            pl.BlockSpec((1, C * DZ * GH, GW), lambda n, j: (n, 0, 0)),
            pl.BlockSpec((1, 1, HB, W), lambda n, j: (n, 0, j, 0)),
        ],
        out_specs=pl.BlockSpec((1, C, HB, W), lambda n, j: (n, 0, j, 0)),
        out_shape=jax.ShapeDtypeStruct((N, C, H, W), jnp.float32),
    )(gridt, guidemap)


# speedup vs baseline: 1.9053x; 1.4107x over previous
import jax
import jax.numpy as jnp
from jax.experimental import pallas as pl


def _fiota(shape, dim):
    return jax.lax.broadcasted_iota(jnp.int32, shape, dim).astype(jnp.float32)


def _dot3(a, b):
    ah = a.astype(jnp.bfloat16)
    al = (a - ah.astype(jnp.float32)).astype(jnp.bfloat16)
    bh = b.astype(jnp.bfloat16)
    bl = (b - bh.astype(jnp.float32)).astype(jnp.bfloat16)
    d = lambda x, y: jnp.dot(x, y, preferred_element_type=jnp.float32)
    return d(ah, bh) + d(al, bh) + d(ah, bl)


def _slice_body(gridt_ref, guide_ref, out_ref, *, C, D, GH, GW, H, W, HB, ZMIN, DZ):
    hb = pl.program_id(1)

    h_idx = hb * HB + _fiota((GW, HB), 1)
    hg = h_idx / (H - 1) * 2.0 - 1.0
    xc = (hg + 1.0) * 0.5 * (GW - 1)
    xrow = _fiota((GW, HB), 0)
    At = jnp.maximum(0.0, 1.0 - jnp.abs(xc - xrow))

    G1 = _dot3(gridt_ref[0], At)
    G1 = G1.reshape(C * DZ, GH, HB)
    G1 = jnp.swapaxes(G1, 1, 2).reshape(C * DZ * HB, GH)

    w_idx = _fiota((GH, W), 1)
    wg = w_idx / (W - 1) * 2.0 - 1.0
    yc = (wg + 1.0) * 0.5 * (GH - 1)
    yrow = _fiota((GH, W), 0)
    Bt = jnp.maximum(0.0, 1.0 - jnp.abs(yc - yrow))

    P = _dot3(G1, Bt)
    P = P.reshape(C, DZ, HB, W)

    g = guide_ref[0, 0]
    zc = (g + 1.0) * 0.5 * (D - 1)
    acc = jnp.zeros((C, HB, W), dtype=jnp.float32)
    for z in range(DZ):
        m = jnp.maximum(0.0, 1.0 - jnp.abs(zc - float(ZMIN + z)))
        acc = acc + P[:, z] * m[None]
    out_ref[0] = acc


def _sc_slice(gridz, gm3, *, C, D, ZMIN, DZ, GH, GW, H, W, h_base=0, HS=None,
              BR=4):
    import functools
    from jax import lax
    from jax.experimental.pallas import tpu as pltpu
    from jax.experimental.pallas import tpu_sc as plsc

    N = gm3.shape[0] // (H * W)
    if HS is None:
        HS = H
    info = plsc.get_sparse_core_info()
    NC, NS = info.num_cores, info.num_subcores
    NW = NC * NS
    hs_w = HS // NW
    bpn = hs_w // BR
    NV = W // 16
    mesh = plsc.VectorSubcoreMesh(core_axis_name="c", subcore_axis_name="s")
    CZ = C * DZ
    zhi = float(D - 1)

    @functools.partial(
        pl.kernel, mesh=mesh,
        out_type=jax.ShapeDtypeStruct((N * C * HS * W,), jnp.float32),
        compiler_params=pltpu.CompilerParams(needs_layout_passes=False),
        scratch_types=[
            pltpu.VMEM((CZ * GH * GW,), jnp.float32),
            pltpu.VMEM((CZ * GH,), jnp.float32),
            pltpu.VMEM((hs_w * W,), jnp.float32),
            pltpu.VMEM((2, C, BR * W), jnp.float32),
            pltpu.VMEM((W,), jnp.int32),
            pltpu.VMEM((W,), jnp.int32),
            pltpu.VMEM((W,), jnp.float32),
            pltpu.VMEM((W,), jnp.float32),
            pltpu.SemaphoreType.DMA((2,)),
        ],
    )
    def sck(gridz_hbm, gm3_hbm, out_hbm, gbuf, rbuf, gdbuf, obuf,
            ybuf, dybuf, wy0buf, wy1buf, sems):
        wid = lax.axis_index("s") * NC + lax.axis_index("c")
        iota16 = lax.broadcasted_iota(jnp.int32, (16,), 0)
        iota16x = iota16 * GW

        def build_tables(j, _):
            wv = (j * 16 + iota16).astype(jnp.float32)
            wgn = wv / (W - 1) * 2.0 - 1.0
            yf = (wgn + 1.0) * 0.5 * (GH - 1)
            y0i = yf.astype(jnp.int32)
            wy1 = yf - y0i.astype(jnp.float32)
            dy = jnp.minimum(y0i + 1, GH - 1) - y0i
            sl = pl.ds(j * 16, 16)
            ybuf[sl] = y0i
            dybuf[sl] = dy
            wy0buf[sl] = 1.0 - wy1
            wy1buf[sl] = wy1
            return 0
        lax.fori_loop(0, NV, build_tables, 0)

        h_w0 = wid * hs_w

        def do_block(n, blk, gblk):
            h0 = h_w0 + blk * BR
            p = lax.rem(gblk, 2)
            @pl.when(gblk >= 2)
            def _():
                for c in range(C):
                    pltpu.make_async_copy(
                        obuf.at[p, c],
                        out_hbm.at[pl.ds(((n * C + c) * HS + h0) * W, BR * W)],
                        sems.at[p]).wait()

            def do_row(r4, _):
                h = h_base + h0 + r4
                hf = (jnp.zeros((16,), jnp.int32) + h).astype(jnp.float32)
                hgn = hf / (H - 1) * 2.0 - 1.0
                xf = (hgn + 1.0) * 0.5 * (GW - 1)
                x0 = xf.astype(jnp.int32)
                wx1 = xf - x0.astype(jnp.float32)
                wx0 = 1.0 - wx1
                dx = jnp.minimum(x0 + 1, GW - 1) - x0
                x0s = x0[0]
                dxs = dx[0]

                @plsc.parallel_loop(0, CZ, unroll=4)
                def fold_x(k):
                    base = (k * GW + x0s) * GH
                    g0 = gbuf[pl.ds(base, 16)]
                    g1 = gbuf[pl.ds(base + dxs * GH, 16)]
                    rbuf[pl.ds(k * 16, 16)] = g0 * wx0 + g1 * wx1

                lrow = blk * BR + r4

                @plsc.parallel_loop(0, NV, unroll=4)
                def do_vec(j):
                    sl = pl.ds(j * 16, 16)
                    g = gdbuf[pl.ds(lrow * W + j * 16, 16)]
                    zc = (g + 1.0) * 0.5 * (D - 1)
                    y0v = ybuf[sl]
                    dyv = dybuf[sl]
                    wy0v = wy0buf[sl]
                    wy1v = wy1buf[sl]
                    y1v = y0v + dyv
                    w0s, w1s = [], []
                    for zr in range(DZ):
                        m = jnp.maximum(0.0, 1.0 - jnp.abs(zc - float(ZMIN + zr)))
                        w0s.append(wy0v * m)
                        w1s.append(wy1v * m)
                    for c in range(C):
                        acc = None
                        for zr in range(DZ):
                            rv = rbuf[pl.ds((c * DZ + zr) * GH, 16)]
                            t = rv.at[y0v].get(mode="promise_in_bounds") * w0s[zr]
                            t = t + rv.at[y1v].get(mode="promise_in_bounds") * w1s[zr]
                            acc = t if acc is None else acc + t
                        obuf[p, c, pl.ds(r4 * W + j * 16, 16)] = acc
                return 0
            lax.fori_loop(0, BR, do_row, 0)

            for c in range(C):
                pltpu.async_copy(
                    obuf.at[p, c],
                    out_hbm.at[pl.ds(((n * C + c) * HS + h0) * W, BR * W)],
                    sems.at[p])

        def do_n(n, _):
            pltpu.sync_copy(gridz_hbm.at[n], gbuf)
            pltpu.sync_copy(
                gm3_hbm.at[pl.ds((n * H + h_base + h_w0) * W, hs_w * W)], gdbuf)
            def blk_body(blk, _):
                do_block(n, blk, n * bpn + blk)
                return 0
            lax.fori_loop(0, bpn, blk_body, 0)
            return 0
        lax.fori_loop(0, N, do_n, 0)

        for p in range(2):
            for c in range(C):
                pltpu.make_async_copy(
                    obuf.at[p, c],
                    out_hbm.at[pl.ds(c * BR * W, BR * W)],
                    sems.at[p]).wait()

    return sck(gridz, gm3)


def kernel(bilateral_grid, guidemap):
    N, C, D, GH, GW = bilateral_grid.shape
    _, _, H, W = guidemap.shape
    ZMIN = (D - 1) // 2
    DZ = D - ZMIN
    gridz = bilateral_grid[:, :, ZMIN:].transpose(0, 1, 2, 4, 3).reshape(N, C * DZ * GH * GW)
    gm3 = guidemap.reshape(N * H * W)
    SC_ROWS = 160
    sc_part = _sc_slice(gridz, gm3, C=C, D=D, ZMIN=ZMIN, DZ=DZ, GH=GH, GW=GW,
                        H=H, W=W, h_base=H - SC_ROWS, HS=SC_ROWS, BR=1)
    sc_part = sc_part.reshape(N, C, SC_ROWS, W)
    tc_part = _kernel_tc(bilateral_grid, guidemap, TCH=H - SC_ROWS, HB=32)
    return jnp.concatenate([tc_part, sc_part], axis=2)


def _kernel_tc(bilateral_grid, guidemap, TCH=None, HB=64):
    N, C, D, GH, GW = bilateral_grid.shape
    _, _, H, W = guidemap.shape
    if TCH is None:
        TCH = H
    ZMIN = (D - 1) // 2
    DZ = D - ZMIN
    gridt = bilateral_grid[:, :, ZMIN:].reshape(N, C * DZ * GH, GW)

    import functools
    body = functools.partial(_slice_body, C=C, D=D, GH=GH, GW=GW, H=H, W=W,
                             HB=HB, ZMIN=ZMIN, DZ=DZ)
    from jax.experimental.pallas import tpu as pltpu
    return pl.pallas_call(
        body,
        grid=(N, TCH // HB),
        compiler_params=pltpu.CompilerParams(
            dimension_semantics=("parallel", "parallel")),
        in_specs=[
            pl.BlockSpec((1, C * DZ * GH, GW), lambda n, j: (n, 0, 0)),
            pl.BlockSpec((1, 1, HB, W), lambda n, j: (n, 0, j, 0)),
        ],
        out_specs=pl.BlockSpec((1, C, HB, W), lambda n, j: (n, 0, j, 0)),
        out_shape=jax.ShapeDtypeStruct((N, C, TCH, W), jnp.float32),
    )(gridt, guidemap)


# speedup vs baseline: 1.9058x; 1.0003x over previous
import jax
import jax.numpy as jnp
from jax.experimental import pallas as pl


def _fiota(shape, dim):
    return jax.lax.broadcasted_iota(jnp.int32, shape, dim).astype(jnp.float32)


def _dot3(a, b):
    ah = a.astype(jnp.bfloat16)
    al = (a - ah.astype(jnp.float32)).astype(jnp.bfloat16)
    bh = b.astype(jnp.bfloat16)
    bl = (b - bh.astype(jnp.float32)).astype(jnp.bfloat16)
    d = lambda x, y: jnp.dot(x, y, preferred_element_type=jnp.float32)
    return d(ah, bh) + d(al, bh) + d(ah, bl)


def _slice_body(gridt_ref, guide_ref, out_ref, *, C, D, GH, GW, H, W, HB, ZMIN, DZ):
    hb = pl.program_id(1)

    h_idx = hb * HB + _fiota((GW, HB), 1)
    hg = h_idx / (H - 1) * 2.0 - 1.0
    xc = (hg + 1.0) * 0.5 * (GW - 1)
    xrow = _fiota((GW, HB), 0)
    At = jnp.maximum(0.0, 1.0 - jnp.abs(xc - xrow))

    G1 = _dot3(gridt_ref[0], At)
    G1 = G1.reshape(C * DZ, GH, HB)
    G1 = jnp.swapaxes(G1, 1, 2).reshape(C * DZ * HB, GH)

    w_idx = _fiota((GH, W), 1)
    wg = w_idx / (W - 1) * 2.0 - 1.0
    yc = (wg + 1.0) * 0.5 * (GH - 1)
    yrow = _fiota((GH, W), 0)
    Bt = jnp.maximum(0.0, 1.0 - jnp.abs(yc - yrow))

    P = _dot3(G1, Bt)
    P = P.reshape(C, DZ, HB, W)

    g = guide_ref[0, 0]
    zc = (g + 1.0) * 0.5 * (D - 1)
    acc = jnp.zeros((C, HB, W), dtype=jnp.float32)
    for z in range(DZ):
        m = jnp.maximum(0.0, 1.0 - jnp.abs(zc - float(ZMIN + z)))
        acc = acc + P[:, z] * m[None]
    out_ref[0] = acc


def _sc_slice(gridz, gm3, *, C, D, ZMIN, DZ, GH, GW, H, W, h_base=0, HS=None,
              BR=4):
    import functools
    from jax import lax
    from jax.experimental.pallas import tpu as pltpu
    from jax.experimental.pallas import tpu_sc as plsc

    N = gm3.shape[0] // (H * W)
    if HS is None:
        HS = H
    info = plsc.get_sparse_core_info()
    NC, NS = info.num_cores, info.num_subcores
    NW = NC * NS
    hs_w = HS // NW
    bpn = hs_w // BR
    NV = W // 16
    mesh = plsc.VectorSubcoreMesh(core_axis_name="c", subcore_axis_name="s")
    CZ = C * DZ
    zhi = float(D - 1)

    @functools.partial(
        pl.kernel, mesh=mesh,
        out_type=jax.ShapeDtypeStruct((N * C * HS * W,), jnp.float32),
        compiler_params=pltpu.CompilerParams(needs_layout_passes=False),
        scratch_types=[
            pltpu.VMEM((CZ * GH * GW,), jnp.float32),
            pltpu.VMEM((CZ * GH,), jnp.float32),
            pltpu.VMEM((hs_w * W,), jnp.float32),
            pltpu.VMEM((2, C, BR * W), jnp.float32),
            pltpu.VMEM((W,), jnp.int32),
            pltpu.VMEM((W,), jnp.int32),
            pltpu.VMEM((W,), jnp.float32),
            pltpu.VMEM((W,), jnp.float32),
            pltpu.SemaphoreType.DMA((2,)),
        ],
    )
    def sck(gridz_hbm, gm3_hbm, out_hbm, gbuf, rbuf, gdbuf, obuf,
            ybuf, dybuf, wy0buf, wy1buf, sems):
        wid = lax.axis_index("s") * NC + lax.axis_index("c")
        iota16 = lax.broadcasted_iota(jnp.int32, (16,), 0)
        iota16x = iota16 * GW

        def build_tables(j, _):
            wv = (j * 16 + iota16).astype(jnp.float32)
            wgn = wv / (W - 1) * 2.0 - 1.0
            yf = (wgn + 1.0) * 0.5 * (GH - 1)
            y0i = yf.astype(jnp.int32)
            wy1 = yf - y0i.astype(jnp.float32)
            dy = jnp.minimum(y0i + 1, GH - 1) - y0i
            sl = pl.ds(j * 16, 16)
            ybuf[sl] = y0i
            dybuf[sl] = dy
            wy0buf[sl] = 1.0 - wy1
            wy1buf[sl] = wy1
            return 0
        lax.fori_loop(0, NV, build_tables, 0)

        h_w0 = wid * hs_w

        def do_block(n, blk, gblk):
            h0 = h_w0 + blk * BR
            p = lax.rem(gblk, 2)
            @pl.when(gblk >= 2)
            def _():
                for c in range(C):
                    pltpu.make_async_copy(
                        obuf.at[p, c],
                        out_hbm.at[pl.ds(((n * C + c) * HS + h0) * W, BR * W)],
                        sems.at[p]).wait()

            def do_row(r4, _):
                h = h_base + h0 + r4
                hf = (jnp.zeros((16,), jnp.int32) + h).astype(jnp.float32)
                hgn = hf / (H - 1) * 2.0 - 1.0
                xf = (hgn + 1.0) * 0.5 * (GW - 1)
                x0 = xf.astype(jnp.int32)
                wx1 = xf - x0.astype(jnp.float32)
                wx0 = 1.0 - wx1
                dx = jnp.minimum(x0 + 1, GW - 1) - x0
                x0s = x0[0]
                dxs = dx[0]

                @plsc.parallel_loop(0, CZ, unroll=4)
                def fold_x(k):
                    base = (k * GW + x0s) * GH
                    g0 = gbuf[pl.ds(base, 16)]
                    g1 = gbuf[pl.ds(base + dxs * GH, 16)]
                    rbuf[pl.ds(k * 16, 16)] = g0 * wx0 + g1 * wx1

                lrow = blk * BR + r4

                @plsc.parallel_loop(0, NV, unroll=4)
                def do_vec(j):
                    sl = pl.ds(j * 16, 16)
                    g = gdbuf[pl.ds(lrow * W + j * 16, 16)]
                    zc = (g + 1.0) * 0.5 * (D - 1)
                    y0v = ybuf[sl]
                    dyv = dybuf[sl]
                    wy0v = wy0buf[sl]
                    wy1v = wy1buf[sl]
                    y1v = y0v + dyv
                    w0s, w1s = [], []
                    for zr in range(DZ):
                        m = jnp.maximum(0.0, 1.0 - jnp.abs(zc - float(ZMIN + zr)))
                        w0s.append(wy0v * m)
                        w1s.append(wy1v * m)
                    for c in range(C):
                        acc = None
                        for zr in range(DZ):
                            rv = rbuf[pl.ds((c * DZ + zr) * GH, 16)]
                            t = rv.at[y0v].get(mode="promise_in_bounds") * w0s[zr]
                            t = t + rv.at[y1v].get(mode="promise_in_bounds") * w1s[zr]
                            acc = t if acc is None else acc + t
                        obuf[p, c, pl.ds(r4 * W + j * 16, 16)] = acc
                return 0
            lax.fori_loop(0, BR, do_row, 0)

            for c in range(C):
                pltpu.async_copy(
                    obuf.at[p, c],
                    out_hbm.at[pl.ds(((n * C + c) * HS + h0) * W, BR * W)],
                    sems.at[p])

        def do_n(n, _):
            pltpu.sync_copy(gridz_hbm.at[n], gbuf)
            pltpu.sync_copy(
                gm3_hbm.at[pl.ds((n * H + h_base + h_w0) * W, hs_w * W)], gdbuf)
            def blk_body(blk, _):
                do_block(n, blk, n * bpn + blk)
                return 0
            lax.fori_loop(0, bpn, blk_body, 0)
            return 0
        lax.fori_loop(0, N, do_n, 0)

        for p in range(2):
            for c in range(C):
                pltpu.make_async_copy(
                    obuf.at[p, c],
                    out_hbm.at[pl.ds(c * BR * W, BR * W)],
                    sems.at[p]).wait()

    return sck(gridz, gm3)


def kernel(bilateral_grid, guidemap):
    N, C, D, GH, GW = bilateral_grid.shape
    _, _, H, W = guidemap.shape
    ZMIN = (D - 1) // 2
    DZ = D - ZMIN
    gridz = bilateral_grid[:, :, ZMIN:].transpose(0, 1, 2, 4, 3).reshape(N, C * DZ * GH * GW)
    gm3 = guidemap.reshape(N * H * W)
    SC_ROWS = 160
    tc_part = _kernel_tc(bilateral_grid, guidemap, TCH=H - SC_ROWS, HB=32)
    sc_part = _sc_slice(gridz, gm3, C=C, D=D, ZMIN=ZMIN, DZ=DZ, GH=GH, GW=GW,
                        H=H, W=W, h_base=H - SC_ROWS, HS=SC_ROWS, BR=1)
    sc_part = sc_part.reshape(N, C, SC_ROWS, W)
    return jnp.concatenate([tc_part, sc_part], axis=2)


def _kernel_tc(bilateral_grid, guidemap, TCH=None, HB=64):
    N, C, D, GH, GW = bilateral_grid.shape
    _, _, H, W = guidemap.shape
    if TCH is None:
        TCH = H
    ZMIN = (D - 1) // 2
    DZ = D - ZMIN
    gridt = bilateral_grid[:, :, ZMIN:].reshape(N, C * DZ * GH, GW)

    import functools
    body = functools.partial(_slice_body, C=C, D=D, GH=GH, GW=GW, H=H, W=W,
                             HB=HB, ZMIN=ZMIN, DZ=DZ)
    from jax.experimental.pallas import tpu as pltpu
    return pl.pallas_call(
        body,
        grid=(N, TCH // HB),
        compiler_params=pltpu.CompilerParams(
            dimension_semantics=("parallel", "parallel")),
        in_specs=[
            pl.BlockSpec((1, C * DZ * GH, GW), lambda n, j: (n, 0, 0)),
            pl.BlockSpec((1, 1, HB, W), lambda n, j: (n, 0, j, 0)),
        ],
        out_specs=pl.BlockSpec((1, C, HB, W), lambda n, j: (n, 0, j, 0)),
        out_shape=jax.ShapeDtypeStruct((N, C, TCH, W), jnp.float32),
    )(gridt, guidemap)


# speedup vs baseline: 2.0763x; 1.0895x over previous
import jax
import jax.numpy as jnp
from jax.experimental import pallas as pl


def _fiota(shape, dim):
    return jax.lax.broadcasted_iota(jnp.int32, shape, dim).astype(jnp.float32)


def _dot3(a, b):
    ah = a.astype(jnp.bfloat16)
    al = (a - ah.astype(jnp.float32)).astype(jnp.bfloat16)
    bh = b.astype(jnp.bfloat16)
    bl = (b - bh.astype(jnp.float32)).astype(jnp.bfloat16)
    d = lambda x, y: jnp.dot(x, y, preferred_element_type=jnp.float32)
    return d(ah, bh) + d(al, bh) + d(ah, bl)


def _slice_body(gridt_ref, guide_ref, out_ref, *, C, D, GH, GW, H, W, HB, ZMIN, DZ):
    hb = pl.program_id(1)

    h_idx = hb * HB + _fiota((GW, HB), 1)
    hg = h_idx / (H - 1) * 2.0 - 1.0
    xc = (hg + 1.0) * 0.5 * (GW - 1)
    xrow = _fiota((GW, HB), 0)
    At = jnp.maximum(0.0, 1.0 - jnp.abs(xc - xrow))

    G1 = _dot3(gridt_ref[0], At)
    G1 = G1.reshape(C * DZ, GH, HB)
    G1 = jnp.swapaxes(G1, 1, 2).reshape(C * DZ * HB, GH)

    w_idx = _fiota((GH, W), 1)
    wg = w_idx / (W - 1) * 2.0 - 1.0
    yc = (wg + 1.0) * 0.5 * (GH - 1)
    yrow = _fiota((GH, W), 0)
    Bt = jnp.maximum(0.0, 1.0 - jnp.abs(yc - yrow))

    P = _dot3(G1, Bt)
    P = P.reshape(C, DZ, HB, W)

    g = guide_ref[0, 0]
    zc = (g + 1.0) * 0.5 * (D - 1)
    acc = jnp.zeros((C, HB, W), dtype=jnp.float32)
    for z in range(DZ):
        m = jnp.maximum(0.0, 1.0 - jnp.abs(zc - float(ZMIN + z)))
        acc = acc + P[:, z] * m[None]
    out_ref[0] = acc


def _sc_slice(gridz, gm3, *, C, D, ZMIN, DZ, GH, GW, H, W, h_base=0, HS=None,
              BR=4):
    import functools
    from jax import lax
    from jax.experimental.pallas import tpu as pltpu
    from jax.experimental.pallas import tpu_sc as plsc

    N = gm3.shape[0] // (H * W)
    if HS is None:
        HS = H
    info = plsc.get_sparse_core_info()
    NC, NS = info.num_cores, info.num_subcores
    NW = NC * NS
    hs_w = HS // NW
    bpn = hs_w // BR
    NV = W // 16
    mesh = plsc.VectorSubcoreMesh(core_axis_name="c", subcore_axis_name="s")
    CZ = C * DZ
    zhi = float(D - 1)

    @functools.partial(
        pl.kernel, mesh=mesh,
        out_type=jax.ShapeDtypeStruct((N * C * HS * W,), jnp.float32),
        compiler_params=pltpu.CompilerParams(needs_layout_passes=False),
        scratch_types=[
            pltpu.VMEM((CZ * GH * GW,), jnp.float32),
            pltpu.VMEM((CZ * GH,), jnp.float32),
            pltpu.VMEM((hs_w * W,), jnp.float32),
            pltpu.VMEM((2, C, BR * W), jnp.float32),
            pltpu.VMEM((W,), jnp.int32),
            pltpu.VMEM((W,), jnp.int32),
            pltpu.VMEM((W,), jnp.float32),
            pltpu.VMEM((W,), jnp.float32),
            pltpu.SemaphoreType.DMA((2,)),
        ],
    )
    def sck(gridz_hbm, gm3_hbm, out_hbm, gbuf, rbuf, gdbuf, obuf,
            ybuf, dybuf, wy0buf, wy1buf, sems):
        wid = lax.axis_index("s") * NC + lax.axis_index("c")
        iota16 = lax.broadcasted_iota(jnp.int32, (16,), 0)
        iota16x = iota16 * GW

        def build_tables(j, _):
            wv = (j * 16 + iota16).astype(jnp.float32)
            wgn = wv / (W - 1) * 2.0 - 1.0
            yf = (wgn + 1.0) * 0.5 * (GH - 1)
            y0i = yf.astype(jnp.int32)
            wy1 = yf - y0i.astype(jnp.float32)
            dy = jnp.minimum(y0i + 1, GH - 1) - y0i
            sl = pl.ds(j * 16, 16)
            ybuf[sl] = y0i
            dybuf[sl] = dy
            wy0buf[sl] = 1.0 - wy1
            wy1buf[sl] = wy1
            return 0
        lax.fori_loop(0, NV, build_tables, 0)

        h_w0 = wid * hs_w

        def do_block(n, blk, gblk):
            h0 = h_w0 + blk * BR
            p = lax.rem(gblk, 2)
            @pl.when(gblk >= 2)
            def _():
                for c in range(C):
                    pltpu.make_async_copy(
                        obuf.at[p, c],
                        out_hbm.at[pl.ds(((n * C + c) * HS + h0) * W, BR * W)],
                        sems.at[p]).wait()

            def do_row(r4, _):
                h = h_base + h0 + r4
                hf = (jnp.zeros((16,), jnp.int32) + h).astype(jnp.float32)
                hgn = hf / (H - 1) * 2.0 - 1.0
                xf = (hgn + 1.0) * 0.5 * (GW - 1)
                x0 = xf.astype(jnp.int32)
                wx1 = xf - x0.astype(jnp.float32)
                wx0 = 1.0 - wx1
                dx = jnp.minimum(x0 + 1, GW - 1) - x0
                x0s = x0[0]
                dxs = dx[0]

                @plsc.parallel_loop(0, CZ, unroll=4)
                def fold_x(k):
                    base = (k * GW + x0s) * GH
                    g0 = gbuf[pl.ds(base, 16)]
                    g1 = gbuf[pl.ds(base + dxs * GH, 16)]
                    rbuf[pl.ds(k * 16, 16)] = g0 * wx0 + g1 * wx1

                lrow = blk * BR + r4

                @plsc.parallel_loop(0, NV, unroll=4)
                def do_vec(j):
                    sl = pl.ds(j * 16, 16)
                    g = gdbuf[pl.ds(lrow * W + j * 16, 16)]
                    zc = (g + 1.0) * 0.5 * (D - 1)
                    y0v = ybuf[sl]
                    dyv = dybuf[sl]
                    wy0v = wy0buf[sl]
                    wy1v = wy1buf[sl]
                    y1v = y0v + dyv
                    w0s, w1s = [], []
                    for zr in range(DZ):
                        m = jnp.maximum(0.0, 1.0 - jnp.abs(zc - float(ZMIN + zr)))
                        w0s.append(wy0v * m)
                        w1s.append(wy1v * m)
                    for c in range(C):
                        acc = None
                        for zr in range(DZ):
                            rv = rbuf[pl.ds((c * DZ + zr) * GH, 16)]
                            t = rv.at[y0v].get(mode="promise_in_bounds") * w0s[zr]
                            t = t + rv.at[y1v].get(mode="promise_in_bounds") * w1s[zr]
                            acc = t if acc is None else acc + t
                        obuf[p, c, pl.ds(r4 * W + j * 16, 16)] = acc
                return 0
            lax.fori_loop(0, BR, do_row, 0)

            for c in range(C):
                pltpu.async_copy(
                    obuf.at[p, c],
                    out_hbm.at[pl.ds(((n * C + c) * HS + h0) * W, BR * W)],
                    sems.at[p])

        def do_n(n, _):
            pltpu.sync_copy(gridz_hbm.at[n], gbuf)
            pltpu.sync_copy(
                gm3_hbm.at[pl.ds((n * H + h_base + h_w0) * W, hs_w * W)], gdbuf)
            def blk_body(blk, _):
                do_block(n, blk, n * bpn + blk)
                return 0
            lax.fori_loop(0, bpn, blk_body, 0)
            return 0
        lax.fori_loop(0, N, do_n, 0)

        for p in range(2):
            for c in range(C):
                pltpu.make_async_copy(
                    obuf.at[p, c],
                    out_hbm.at[pl.ds(c * BR * W, BR * W)],
                    sems.at[p]).wait()

    return sck(gridz, gm3)


def kernel(bilateral_grid, guidemap):
    N, C, D, GH, GW = bilateral_grid.shape
    _, _, H, W = guidemap.shape
    ZMIN = (D - 1) // 2
    DZ = D - ZMIN
    gridz = bilateral_grid[:, :, ZMIN:].transpose(0, 1, 2, 4, 3).reshape(N, C * DZ * GH * GW)
    gm3 = guidemap.reshape(N * H * W)
    SC_ROWS = 128
    tc_part = _kernel_tc(bilateral_grid, guidemap, TCH=H - SC_ROWS, HB=64)
    sc_part = _sc_slice(gridz, gm3, C=C, D=D, ZMIN=ZMIN, DZ=DZ, GH=GH, GW=GW,
                        H=H, W=W, h_base=H - SC_ROWS, HS=SC_ROWS, BR=4)
    sc_part = sc_part.reshape(N, C, SC_ROWS, W)
    return jnp.concatenate([tc_part, sc_part], axis=2)


def _kernel_tc(bilateral_grid, guidemap, TCH=None, HB=64):
    N, C, D, GH, GW = bilateral_grid.shape
    _, _, H, W = guidemap.shape
    if TCH is None:
        TCH = H
    ZMIN = (D - 1) // 2
    DZ = D - ZMIN
    gridt = bilateral_grid[:, :, ZMIN:].reshape(N, C * DZ * GH, GW)

    import functools
    body = functools.partial(_slice_body, C=C, D=D, GH=GH, GW=GW, H=H, W=W,
                             HB=HB, ZMIN=ZMIN, DZ=DZ)
    from jax.experimental.pallas import tpu as pltpu
    return pl.pallas_call(
        body,
        grid=(N, TCH // HB),
        compiler_params=pltpu.CompilerParams(
            dimension_semantics=("parallel", "parallel")),
        in_specs=[
            pl.BlockSpec((1, C * DZ * GH, GW), lambda n, j: (n, 0, 0)),
            pl.BlockSpec((1, 1, HB, W), lambda n, j: (n, 0, j, 0)),
        ],
        out_specs=pl.BlockSpec((1, C, HB, W), lambda n, j: (n, 0, j, 0)),
        out_shape=jax.ShapeDtypeStruct((N, C, TCH, W), jnp.float32),
    )(gridt, guidemap)
